# baseline (speedup 1.0000x reference)
import functools

import jax
import jax.numpy as jnp
from jax import lax
from jax.experimental import pallas as pl
from jax.experimental.pallas import tpu as pltpu
from jax.experimental.pallas import tpu_sc as plsc

BN_EPS = 1e-5
NC = 2
NS = 16
NW = NC * NS
CHUNK = 256


def _proj_body(x_ref, w_ref, o_ref):
    o_ref[...] = jnp.dot(x_ref[...], w_ref[...],
                         preferred_element_type=jnp.float32)


def _layer0_body(c0_ref, p_ref, pa_ref, pb_ref, bb0_ref, w20_ref, u0_ref,
                 w11_ref, o_ref):
    z = (c0_ref[...] * p_ref[...] + pa_ref[0] + pb_ref[0]
         + bb0_ref[...])
    a = jnp.maximum(z, 0.0)
    h = jnp.maximum(
        jnp.dot(a, w20_ref[...], preferred_element_type=jnp.float32)
        + u0_ref[...], 0.0)
    o_ref[...] = jnp.dot(h, w11_ref[...], preferred_element_type=jnp.float32)


def _layer1_body(n_ref, c1_ref, p_ref, pa_ref, pb_ref, bb1_ref, w21_ref,
                 v1_ref, o_ref, acc_ref):
    i = pl.program_id(0)

    @pl.when(i == 0)
    def _():
        acc_ref[...] = jnp.zeros_like(acc_ref)

    z = (c1_ref[...] * p_ref[...] + pa_ref[0] + pb_ref[0]
         + bb1_ref[...])
    q = jnp.maximum(z, 0.0)
    acc_ref[...] += jnp.sum(q, axis=0, keepdims=True)

    @pl.when(i == pl.num_programs(0) - 1)
    def _():
        mean = acc_ref[...] * n_ref[...]
        o_ref[...] = jnp.dot(mean, w21_ref[...],
                             preferred_element_type=jnp.float32) + v1_ref[...]


def _make_agg(n_pad, feat, k_chunks):
    nbuf = 4
    rows_per_tile = n_pad // NS
    assert rows_per_tile % 16 == 0 and k_chunks % nbuf == 0
    mesh = plsc.VectorSubcoreMesh(core_axis_name="c", subcore_axis_name="s")

    @functools.partial(
        pl.kernel,
        out_type=jax.ShapeDtypeStruct((NC, n_pad, feat), jnp.float32),
        mesh=mesh,
        scratch_types=[
            pltpu.VMEM((k_chunks, CHUNK), jnp.int32),
            pltpu.VMEM((k_chunks, CHUNK), jnp.int32),
            pltpu.VMEM((nbuf, CHUNK, feat), jnp.float32),
            pltpu.VMEM((16, feat), jnp.float32),
            pltpu.VMEM_SHARED((n_pad, feat), jnp.float32),
        ] + [pltpu.SemaphoreType.DMA] * (2 * nbuf),
        compiler_params=pltpu.CompilerParams(use_tc_tiling_on_sc=False),
    )
    def agg(p_hbm, src_hbm, dst_hbm, out_hbm, src_v, dst_v, rows_v, zb, acc,
            *sems):
        cid = lax.axis_index("c")
        sid = lax.axis_index("s")
        wid = cid * NS + sid
        base = sid * rows_per_tile

        def _zrow(r, carry):
            for c in range(feat // 16):
                zb[r, pl.ds(c * 16, 16)] = jnp.zeros((16,), jnp.float32)
            return carry
        lax.fori_loop(0, 16, _zrow, 0)

        def _zcopy(t, carry):
            pltpu.sync_copy(zb, acc.at[pl.ds(base + t * 16, 16)])
            return carry
        lax.fori_loop(0, rows_per_tile // 16, _zcopy, 0)

        pltpu.sync_copy(src_hbm.at[wid], src_v)
        pltpu.sync_copy(dst_hbm.at[wid], dst_v)

        plsc.subcore_barrier()

        gsems = sems[:nbuf]
        ssems = sems[nbuf:]

        def _start(j, b):
            pltpu.async_copy(p_hbm.at[src_v.at[j]], rows_v.at[b], gsems[b])

        for b in range(nbuf):
            _start(b, b)

        def _step(jo, carry):
            for b in range(nbuf):
                j = jo * nbuf + b
                pltpu.make_async_copy(
                    p_hbm.at[src_v.at[j]], rows_v.at[b], gsems[b]).wait()

                pltpu.async_copy(rows_v.at[b], acc.at[dst_v.at[j]], ssems[b],
                                 add=True)

                bp = (b - 1) % nbuf
                jp = j - 1

                @pl.when((jp >= 0) & (jp + nbuf < k_chunks))
                def _():
                    pltpu.make_async_copy(
                        rows_v.at[bp], acc.at[dst_v.at[jp]], ssems[bp]).wait()
                    _start(jp + nbuf, bp)
            return carry
        lax.fori_loop(0, k_chunks // nbuf, _step, 0)

        for b in range(nbuf):
            j = k_chunks - nbuf + b
            pltpu.make_async_copy(
                rows_v.at[b], acc.at[dst_v.at[j]], ssems[b]).wait()

        plsc.subcore_barrier()

        pltpu.sync_copy(acc.at[pl.ds(base, rows_per_tile)],
                        out_hbm.at[cid, pl.ds(base, rows_per_tile)])

    return agg


def kernel(x, edge_index, W1_0, b1_0, W2_0, b2_0, eps0, gamma0, beta0,
           W1_1, b1_1, W2_1, b2_1, eps1, gamma1, beta1):
    n, d_in = x.shape
    h0 = W1_0.shape[1]
    h1 = W1_1.shape[1]
    e = edge_index.shape[1]

    n_pad = ((n + 1 + 255) // 256) * 256
    k_chunks = -(-e // (NW * CHUNK))
    k_chunks = ((k_chunks + 3) // 4) * 4
    e_pad = NW * k_chunks * CHUNK

    src = edge_index[0]
    dst = edge_index[1]
    src3 = jnp.pad(src, (0, e_pad - e)).reshape(NW, k_chunks, CHUNK)
    dst3 = jnp.pad(dst, (0, e_pad - e),
                   constant_values=n).reshape(NW, k_chunks, CHUNK)

    inv = 1.0 / jnp.sqrt(1.0 + BN_EPS)
    g0 = gamma0 * inv
    w20s = W2_0 * g0[None, :]
    u0 = (g0 * b2_0 + beta0).reshape(1, h0)
    g1 = gamma1 * inv
    w21s = W2_1 * g1[None, :]
    v1 = (g1 * b2_1 + beta1).reshape(1, h1)
    bb0 = b1_0.reshape(1, h0)
    bb1 = b1_1.reshape(1, h1)
    c0 = (1.0 + eps0).reshape(1, 1)
    c1 = (1.0 + eps1).reshape(1, 1)

    blk = 2000
    grid = n // blk

    p0 = pl.pallas_call(
        _proj_body,
        grid=(grid,),
        in_specs=[pl.BlockSpec((blk, d_in), lambda i: (i, 0)),
                  pl.BlockSpec((d_in, h0), lambda i: (0, 0))],
        out_specs=pl.BlockSpec((blk, h0), lambda i: (i, 0)),
        out_shape=jax.ShapeDtypeStruct((n, h0), jnp.float32),
    )(x, W1_0)

    agg0 = _make_agg(n_pad, h0, k_chunks)(p0, src3, dst3)

    p1 = pl.pallas_call(
        _layer0_body,
        grid=(grid,),
        in_specs=[pl.BlockSpec((1, 1), lambda i: (0, 0)),
                  pl.BlockSpec((blk, h0), lambda i: (i, 0)),
                  pl.BlockSpec((1, blk, h0), lambda i: (0, i, 0)),
                  pl.BlockSpec((1, blk, h0), lambda i: (1, i, 0)),
                  pl.BlockSpec((1, h0), lambda i: (0, 0)),
                  pl.BlockSpec((h0, h0), lambda i: (0, 0)),
                  pl.BlockSpec((1, h0), lambda i: (0, 0)),
                  pl.BlockSpec((h0, h1), lambda i: (0, 0))],
        out_specs=pl.BlockSpec((blk, h1), lambda i: (i, 0)),
        out_shape=jax.ShapeDtypeStruct((n, h1), jnp.float32),
    )(c0, p0, agg0, agg0, bb0, w20s, u0, W1_1)

    agg1 = _make_agg(n_pad, h1, k_chunks)(p1, src3, dst3)

    n_inv = jnp.full((1, 1), 1.0 / n, jnp.float32)
    out = pl.pallas_call(
        _layer1_body,
        grid=(grid,),
        in_specs=[pl.BlockSpec((1, 1), lambda i: (0, 0)),
                  pl.BlockSpec((1, 1), lambda i: (0, 0)),
                  pl.BlockSpec((blk, h1), lambda i: (i, 0)),
                  pl.BlockSpec((1, blk, h1), lambda i: (0, i, 0)),
                  pl.BlockSpec((1, blk, h1), lambda i: (1, i, 0)),
                  pl.BlockSpec((1, h1), lambda i: (0, 0)),
                  pl.BlockSpec((h1, h1), lambda i: (0, 0)),
                  pl.BlockSpec((1, h1), lambda i: (0, 0))],
        out_specs=pl.BlockSpec((1, h1), lambda i: (0, 0)),
        out_shape=jax.ShapeDtypeStruct((1, h1), jnp.float32),
        scratch_shapes=[pltpu.VMEM((1, h1), jnp.float32)],
    )(n_inv, c1, p1, agg1, agg1, bb1, w21s, v1)

    return out


# speedup vs baseline: 1.0022x; 1.0022x over previous
import functools

import jax
import jax.numpy as jnp
from jax import lax
from jax.experimental import pallas as pl
from jax.experimental.pallas import tpu as pltpu
from jax.experimental.pallas import tpu_sc as plsc

BN_EPS = 1e-5
NC = 2
NS = 16
NW = NC * NS
CHUNK = 128


def _proj_body(x_ref, w_ref, o_ref):
    o_ref[...] = jnp.dot(x_ref[...], w_ref[...],
                         preferred_element_type=jnp.float32)


def _layer0_body(c0_ref, p_ref, pa_ref, pb_ref, bb0_ref, w20_ref, u0_ref,
                 w11_ref, o_ref):
    z = (c0_ref[...] * p_ref[...] + pa_ref[0] + pb_ref[0]
         + bb0_ref[...])
    a = jnp.maximum(z, 0.0)
    h = jnp.maximum(
        jnp.dot(a, w20_ref[...], preferred_element_type=jnp.float32)
        + u0_ref[...], 0.0)
    o_ref[...] = jnp.dot(h, w11_ref[...], preferred_element_type=jnp.float32)


def _layer1_body(n_ref, c1_ref, p_ref, pa_ref, pb_ref, bb1_ref, w21_ref,
                 v1_ref, o_ref, acc_ref):
    i = pl.program_id(0)

    @pl.when(i == 0)
    def _():
        acc_ref[...] = jnp.zeros_like(acc_ref)

    z = (c1_ref[...] * p_ref[...] + pa_ref[0] + pb_ref[0]
         + bb1_ref[...])
    q = jnp.maximum(z, 0.0)
    acc_ref[...] += jnp.sum(q, axis=0, keepdims=True)

    @pl.when(i == pl.num_programs(0) - 1)
    def _():
        mean = acc_ref[...] * n_ref[...]
        o_ref[...] = jnp.dot(mean, w21_ref[...],
                             preferred_element_type=jnp.float32) + v1_ref[...]


def _make_agg(n_pad, feat, k_chunks):
    nbuf = 8
    rows_per_tile = n_pad // NS
    assert rows_per_tile % 16 == 0 and k_chunks % nbuf == 0
    mesh = plsc.VectorSubcoreMesh(core_axis_name="c", subcore_axis_name="s")

    @functools.partial(
        pl.kernel,
        out_type=jax.ShapeDtypeStruct((NC, n_pad, feat), jnp.float32),
        mesh=mesh,
        scratch_types=[
            pltpu.VMEM((k_chunks, CHUNK), jnp.int32),
            pltpu.VMEM((k_chunks, CHUNK), jnp.int32),
            pltpu.VMEM((nbuf, CHUNK, feat), jnp.float32),
            pltpu.VMEM((16, feat), jnp.float32),
            pltpu.VMEM_SHARED((n_pad, feat), jnp.float32),
        ] + [pltpu.SemaphoreType.DMA] * (2 * nbuf),
        compiler_params=pltpu.CompilerParams(use_tc_tiling_on_sc=False),
    )
    def agg(p_hbm, src_hbm, dst_hbm, out_hbm, src_v, dst_v, rows_v, zb, acc,
            *sems):
        cid = lax.axis_index("c")
        sid = lax.axis_index("s")
        wid = cid * NS + sid
        base = sid * rows_per_tile

        def _zrow(r, carry):
            for c in range(feat // 16):
                zb[r, pl.ds(c * 16, 16)] = jnp.zeros((16,), jnp.float32)
            return carry
        lax.fori_loop(0, 16, _zrow, 0)

        def _zcopy(t, carry):
            pltpu.sync_copy(zb, acc.at[pl.ds(base + t * 16, 16)])
            return carry
        lax.fori_loop(0, rows_per_tile // 16, _zcopy, 0)

        pltpu.sync_copy(src_hbm.at[wid], src_v)
        pltpu.sync_copy(dst_hbm.at[wid], dst_v)

        plsc.subcore_barrier()

        gsems = sems[:nbuf]
        ssems = sems[nbuf:]

        def _start(j, b):
            pltpu.async_copy(p_hbm.at[src_v.at[j]], rows_v.at[b], gsems[b])

        for b in range(nbuf):
            _start(b, b)

        def _step(jo, carry):
            for b in range(nbuf):
                j = jo * nbuf + b
                pltpu.make_async_copy(
                    p_hbm.at[src_v.at[j]], rows_v.at[b], gsems[b]).wait()

                pltpu.async_copy(rows_v.at[b], acc.at[dst_v.at[j]], ssems[b],
                                 add=True)

                bp = (b - 1) % nbuf
                jp = j - 1

                @pl.when((jp >= 0) & (jp + nbuf < k_chunks))
                def _():
                    pltpu.make_async_copy(
                        rows_v.at[bp], acc.at[dst_v.at[jp]], ssems[bp]).wait()
                    _start(jp + nbuf, bp)
            return carry
        lax.fori_loop(0, k_chunks // nbuf, _step, 0)

        for b in range(nbuf):
            j = k_chunks - nbuf + b
            pltpu.make_async_copy(
                rows_v.at[b], acc.at[dst_v.at[j]], ssems[b]).wait()

        plsc.subcore_barrier()

        pltpu.sync_copy(acc.at[pl.ds(base, rows_per_tile)],
                        out_hbm.at[cid, pl.ds(base, rows_per_tile)])

    return agg


def kernel(x, edge_index, W1_0, b1_0, W2_0, b2_0, eps0, gamma0, beta0,
           W1_1, b1_1, W2_1, b2_1, eps1, gamma1, beta1):
    n, d_in = x.shape
    h0 = W1_0.shape[1]
    h1 = W1_1.shape[1]
    e = edge_index.shape[1]

    n_pad = ((n + 1 + 255) // 256) * 256
    k_chunks = -(-e // (NW * CHUNK))
    k_chunks = ((k_chunks + 7) // 8) * 8
    e_pad = NW * k_chunks * CHUNK

    src = edge_index[0]
    dst = edge_index[1]
    src3 = jnp.pad(src, (0, e_pad - e)).reshape(NW, k_chunks, CHUNK)
    dst3 = jnp.pad(dst, (0, e_pad - e),
                   constant_values=n).reshape(NW, k_chunks, CHUNK)

    inv = 1.0 / jnp.sqrt(1.0 + BN_EPS)
    g0 = gamma0 * inv
    w20s = W2_0 * g0[None, :]
    u0 = (g0 * b2_0 + beta0).reshape(1, h0)
    g1 = gamma1 * inv
    w21s = W2_1 * g1[None, :]
    v1 = (g1 * b2_1 + beta1).reshape(1, h1)
    bb0 = b1_0.reshape(1, h0)
    bb1 = b1_1.reshape(1, h1)
    c0 = (1.0 + eps0).reshape(1, 1)
    c1 = (1.0 + eps1).reshape(1, 1)

    blk = 2000
    grid = n // blk

    p0 = pl.pallas_call(
        _proj_body,
        grid=(grid,),
        in_specs=[pl.BlockSpec((blk, d_in), lambda i: (i, 0)),
                  pl.BlockSpec((d_in, h0), lambda i: (0, 0))],
        out_specs=pl.BlockSpec((blk, h0), lambda i: (i, 0)),
        out_shape=jax.ShapeDtypeStruct((n, h0), jnp.float32),
    )(x, W1_0)

    agg0 = _make_agg(n_pad, h0, k_chunks)(p0, src3, dst3)

    p1 = pl.pallas_call(
        _layer0_body,
        grid=(grid,),
        in_specs=[pl.BlockSpec((1, 1), lambda i: (0, 0)),
                  pl.BlockSpec((blk, h0), lambda i: (i, 0)),
                  pl.BlockSpec((1, blk, h0), lambda i: (0, i, 0)),
                  pl.BlockSpec((1, blk, h0), lambda i: (1, i, 0)),
                  pl.BlockSpec((1, h0), lambda i: (0, 0)),
                  pl.BlockSpec((h0, h0), lambda i: (0, 0)),
                  pl.BlockSpec((1, h0), lambda i: (0, 0)),
                  pl.BlockSpec((h0, h1), lambda i: (0, 0))],
        out_specs=pl.BlockSpec((blk, h1), lambda i: (i, 0)),
        out_shape=jax.ShapeDtypeStruct((n, h1), jnp.float32),
    )(c0, p0, agg0, agg0, bb0, w20s, u0, W1_1)

    agg1 = _make_agg(n_pad, h1, k_chunks)(p1, src3, dst3)

    n_inv = jnp.full((1, 1), 1.0 / n, jnp.float32)
    out = pl.pallas_call(
        _layer1_body,
        grid=(grid,),
        in_specs=[pl.BlockSpec((1, 1), lambda i: (0, 0)),
                  pl.BlockSpec((1, 1), lambda i: (0, 0)),
                  pl.BlockSpec((blk, h1), lambda i: (i, 0)),
                  pl.BlockSpec((1, blk, h1), lambda i: (0, i, 0)),
                  pl.BlockSpec((1, blk, h1), lambda i: (1, i, 0)),
                  pl.BlockSpec((1, h1), lambda i: (0, 0)),
                  pl.BlockSpec((h1, h1), lambda i: (0, 0)),
                  pl.BlockSpec((1, h1), lambda i: (0, 0))],
        out_specs=pl.BlockSpec((1, h1), lambda i: (0, 0)),
        out_shape=jax.ShapeDtypeStruct((1, h1), jnp.float32),
        scratch_shapes=[pltpu.VMEM((1, h1), jnp.float32)],
    )(n_inv, c1, p1, agg1, agg1, bb1, w21s, v1)

    return out


# speedup vs baseline: 1.1026x; 1.1002x over previous
import functools

import jax
import jax.numpy as jnp
from jax import lax
from jax.experimental import pallas as pl
from jax.experimental.pallas import tpu as pltpu
from jax.experimental.pallas import tpu_sc as plsc

BN_EPS = 1e-5
NC = 2
NS = 16
NW = NC * NS
CHUNK = 128


def _proj_body(x_ref, w_ref, o_ref):
    o_ref[...] = jnp.dot(x_ref[...], w_ref[...],
                         preferred_element_type=jnp.float32)


def _layer0_body(c0_ref, p_ref, pa_ref, pb_ref, bb0_ref, w20_ref, u0_ref,
                 w11_ref, o_ref):
    z = c0_ref[...] * p_ref[...] + pa_ref[...] + pb_ref[...] + bb0_ref[...]
    a = jnp.maximum(z, 0.0)
    h = jnp.maximum(
        jnp.dot(a, w20_ref[...], preferred_element_type=jnp.float32)
        + u0_ref[...], 0.0)
    o_ref[...] = jnp.dot(h, w11_ref[...], preferred_element_type=jnp.float32)


def _layer1_body(n_ref, c1_ref, p_ref, pa_ref, pb_ref, bb1_ref, w21_ref,
                 v1_ref, o_ref, acc_ref):
    i = pl.program_id(0)

    @pl.when(i == 0)
    def _():
        acc_ref[...] = jnp.zeros_like(acc_ref)

    z = c1_ref[...] * p_ref[...] + pa_ref[...] + pb_ref[...] + bb1_ref[...]
    q = jnp.maximum(z, 0.0)
    acc_ref[...] += jnp.sum(q, axis=0, keepdims=True)

    @pl.when(i == pl.num_programs(0) - 1)
    def _():
        mean = acc_ref[...] * n_ref[...]
        o_ref[...] = jnp.dot(mean, w21_ref[...],
                             preferred_element_type=jnp.float32) + v1_ref[...]


def _make_agg(n_pad, feat, k_chunks):
    nbuf = 8
    rows_per_tile = n_pad // NS
    assert rows_per_tile % 16 == 0 and k_chunks % nbuf == 0
    mesh = plsc.VectorSubcoreMesh(core_axis_name="c", subcore_axis_name="s")

    @functools.partial(
        pl.kernel,
        out_type=jax.ShapeDtypeStruct((NC, n_pad, feat), jnp.float32),
        mesh=mesh,
        scratch_types=[
            pltpu.VMEM((k_chunks, CHUNK), jnp.int32),
            pltpu.VMEM((k_chunks, CHUNK), jnp.int32),
            pltpu.VMEM((nbuf, CHUNK, feat), jnp.float32),
            pltpu.VMEM((16, feat), jnp.float32),
            pltpu.VMEM_SHARED((n_pad, feat), jnp.float32),
        ] + [pltpu.SemaphoreType.DMA] * (2 * nbuf),
        compiler_params=pltpu.CompilerParams(use_tc_tiling_on_sc=False),
    )
    def agg(p_hbm, src_hbm, dst_hbm, out_hbm, src_v, dst_v, rows_v, zb, acc,
            *sems):
        cid = lax.axis_index("c")
        sid = lax.axis_index("s")
        wid = cid * NS + sid
        base = sid * rows_per_tile

        def _zrow(r, carry):
            for c in range(feat // 16):
                zb[r, pl.ds(c * 16, 16)] = jnp.zeros((16,), jnp.float32)
            return carry
        lax.fori_loop(0, 16, _zrow, 0)

        def _zcopy(t, carry):
            pltpu.sync_copy(zb, acc.at[pl.ds(base + t * 16, 16)])
            return carry
        lax.fori_loop(0, rows_per_tile // 16, _zcopy, 0)

        pltpu.sync_copy(src_hbm.at[wid], src_v)
        pltpu.sync_copy(dst_hbm.at[wid], dst_v)

        plsc.subcore_barrier()

        gsems = sems[:nbuf]
        ssems = sems[nbuf:]

        def _start(j, b):
            pltpu.async_copy(p_hbm.at[src_v.at[j]], rows_v.at[b], gsems[b])

        for b in range(nbuf):
            _start(b, b)

        def _step(jo, carry):
            for b in range(nbuf):
                j = jo * nbuf + b
                pltpu.make_async_copy(
                    p_hbm.at[src_v.at[j]], rows_v.at[b], gsems[b]).wait()

                pltpu.async_copy(rows_v.at[b], acc.at[dst_v.at[j]], ssems[b],
                                 add=True)

                bp = (b - 1) % nbuf
                jp = j - 1

                @pl.when((jp >= 0) & (jp + nbuf < k_chunks))
                def _():
                    pltpu.make_async_copy(
                        rows_v.at[bp], acc.at[dst_v.at[jp]], ssems[bp]).wait()
                    _start(jp + nbuf, bp)
            return carry
        lax.fori_loop(0, k_chunks // nbuf, _step, 0)

        for b in range(nbuf):
            j = k_chunks - nbuf + b
            pltpu.make_async_copy(
                rows_v.at[b], acc.at[dst_v.at[j]], ssems[b]).wait()

        plsc.subcore_barrier()

        pltpu.sync_copy(acc.at[pl.ds(base, rows_per_tile)],
                        out_hbm.at[cid, pl.ds(base, rows_per_tile)])

    return agg


def kernel(x, edge_index, W1_0, b1_0, W2_0, b2_0, eps0, gamma0, beta0,
           W1_1, b1_1, W2_1, b2_1, eps1, gamma1, beta1):
    n, d_in = x.shape
    h0 = W1_0.shape[1]
    h1 = W1_1.shape[1]
    e = edge_index.shape[1]

    n_pad = ((n + 1 + 255) // 256) * 256
    k_chunks = -(-e // (NW * CHUNK))
    k_chunks = ((k_chunks + 7) // 8) * 8
    e_pad = NW * k_chunks * CHUNK

    src = edge_index[0]
    dst = edge_index[1]
    src3 = jnp.pad(src, (0, e_pad - e)).reshape(NW, k_chunks, CHUNK)
    dst3 = jnp.pad(dst, (0, e_pad - e),
                   constant_values=n).reshape(NW, k_chunks, CHUNK)

    inv = 1.0 / jnp.sqrt(1.0 + BN_EPS)
    g0 = gamma0 * inv
    w20s = W2_0 * g0[None, :]
    u0 = (g0 * b2_0 + beta0).reshape(1, h0)
    g1 = gamma1 * inv
    w21s = W2_1 * g1[None, :]
    v1 = (g1 * b2_1 + beta1).reshape(1, h1)
    bb0 = b1_0.reshape(1, h0)
    bb1 = b1_1.reshape(1, h1)
    c0 = (1.0 + eps0).reshape(1, 1)
    c1 = (1.0 + eps1).reshape(1, 1)

    blk = 2000
    grid = n // blk

    p0 = pl.pallas_call(
        _proj_body,
        grid=(grid,),
        in_specs=[pl.BlockSpec((blk, d_in), lambda i: (i, 0)),
                  pl.BlockSpec((d_in, h0), lambda i: (0, 0))],
        out_specs=pl.BlockSpec((blk, h0), lambda i: (i, 0)),
        out_shape=jax.ShapeDtypeStruct((n, h0), jnp.float32),
    )(x, W1_0)

    agg0 = _make_agg(n_pad, h0, k_chunks)(p0, src3, dst3)
    pa0 = agg0[0, :n]
    pb0 = agg0[1, :n]

    p1 = pl.pallas_call(
        _layer0_body,
        grid=(grid,),
        in_specs=[pl.BlockSpec((1, 1), lambda i: (0, 0)),
                  pl.BlockSpec((blk, h0), lambda i: (i, 0)),
                  pl.BlockSpec((blk, h0), lambda i: (i, 0)),
                  pl.BlockSpec((blk, h0), lambda i: (i, 0)),
                  pl.BlockSpec((1, h0), lambda i: (0, 0)),
                  pl.BlockSpec((h0, h0), lambda i: (0, 0)),
                  pl.BlockSpec((1, h0), lambda i: (0, 0)),
                  pl.BlockSpec((h0, h1), lambda i: (0, 0))],
        out_specs=pl.BlockSpec((blk, h1), lambda i: (i, 0)),
        out_shape=jax.ShapeDtypeStruct((n, h1), jnp.float32),
    )(c0, p0, pa0, pb0, bb0, w20s, u0, W1_1)

    agg1 = _make_agg(n_pad, h1, k_chunks)(p1, src3, dst3)
    pa1 = agg1[0, :n]
    pb1 = agg1[1, :n]

    n_inv = jnp.full((1, 1), 1.0 / n, jnp.float32)
    out = pl.pallas_call(
        _layer1_body,
        grid=(grid,),
        in_specs=[pl.BlockSpec((1, 1), lambda i: (0, 0)),
                  pl.BlockSpec((1, 1), lambda i: (0, 0)),
                  pl.BlockSpec((blk, h1), lambda i: (i, 0)),
                  pl.BlockSpec((blk, h1), lambda i: (i, 0)),
                  pl.BlockSpec((blk, h1), lambda i: (i, 0)),
                  pl.BlockSpec((1, h1), lambda i: (0, 0)),
                  pl.BlockSpec((h1, h1), lambda i: (0, 0)),
                  pl.BlockSpec((1, h1), lambda i: (0, 0))],
        out_specs=pl.BlockSpec((1, h1), lambda i: (0, 0)),
        out_shape=jax.ShapeDtypeStruct((1, h1), jnp.float32),
        scratch_shapes=[pltpu.VMEM((1, h1), jnp.float32)],
    )(n_inv, c1, p1, pa1, pb1, bb1, w21s, v1)

    return out


# speedup vs baseline: 2.1856x; 1.9822x over previous
import functools

import jax
import jax.numpy as jnp
from jax import lax
from jax.experimental import pallas as pl
from jax.experimental.pallas import tpu as pltpu
from jax.experimental.pallas import tpu_sc as plsc

BN_EPS = 1e-5
NC = 2
NS = 16
NW = NC * NS
CHUNK = 128


def _proj_body(x_ref, w_ref, o_ref):
    o_ref[...] = jnp.dot(x_ref[...], w_ref[...],
                         preferred_element_type=jnp.float32)


def _layer0_body(c0_ref, p_ref, pa_ref, pb_ref, bb0_ref, w20_ref, u0_ref,
                 w11_ref, o_ref):
    z = c0_ref[...] * p_ref[...] + pa_ref[...] + pb_ref[...] + bb0_ref[...]
    a = jnp.maximum(z, 0.0)
    h = jnp.maximum(
        jnp.dot(a, w20_ref[...], preferred_element_type=jnp.float32)
        + u0_ref[...], 0.0)
    o_ref[...] = jnp.dot(h, w11_ref[...], preferred_element_type=jnp.float32)


def _layer1_body(n_ref, c1_ref, p_ref, pa_ref, pb_ref, bb1_ref, w21_ref,
                 v1_ref, o_ref, acc_ref):
    i = pl.program_id(0)

    @pl.when(i == 0)
    def _():
        acc_ref[...] = jnp.zeros_like(acc_ref)

    z = c1_ref[...] * p_ref[...] + pa_ref[...] + pb_ref[...] + bb1_ref[...]
    q = jnp.maximum(z, 0.0)
    acc_ref[...] += jnp.sum(q, axis=0, keepdims=True)

    @pl.when(i == pl.num_programs(0) - 1)
    def _():
        mean = acc_ref[...] * n_ref[...]
        o_ref[...] = jnp.dot(mean, w21_ref[...],
                             preferred_element_type=jnp.float32) + v1_ref[...]


def _make_agg(n_pad, n_rows, feat, k_chunks):
    nbuf = 2
    rows_per_tile = n_pad // NS

    rows_stage = n_rows // NS
    assert rows_per_tile % 16 == 0 and k_chunks % nbuf == 0
    assert n_rows % NS == 0
    mesh = plsc.VectorSubcoreMesh(core_axis_name="c", subcore_axis_name="s")

    @functools.partial(
        pl.kernel,
        out_type=jax.ShapeDtypeStruct((NC, n_pad, feat), jnp.float32),
        mesh=mesh,
        scratch_types=[
            pltpu.VMEM((k_chunks, CHUNK), jnp.int32),
            pltpu.VMEM((k_chunks, CHUNK), jnp.int32),
            pltpu.VMEM((nbuf, CHUNK, feat), jnp.float32),
            pltpu.VMEM((16, feat), jnp.float32),
            pltpu.VMEM_SHARED((n_pad, feat), jnp.float32),
            pltpu.VMEM_SHARED((n_rows, feat), jnp.float32),
        ] + [pltpu.SemaphoreType.DMA] * (2 * nbuf),
        compiler_params=pltpu.CompilerParams(use_tc_tiling_on_sc=False),
    )
    def agg(p_hbm, src_hbm, dst_hbm, out_hbm, src_v, dst_v, rows_v, zb, acc,
            p_s, *sems):
        cid = lax.axis_index("c")
        sid = lax.axis_index("s")
        wid = cid * NS + sid
        base = sid * rows_per_tile

        def _zrow(r, carry):
            for c in range(feat // 16):
                zb[r, pl.ds(c * 16, 16)] = jnp.zeros((16,), jnp.float32)
            return carry
        lax.fori_loop(0, 16, _zrow, 0)

        def _zcopy(t, carry):
            pltpu.sync_copy(zb, acc.at[pl.ds(base + t * 16, 16)])
            return carry
        lax.fori_loop(0, rows_per_tile // 16, _zcopy, 0)

        pltpu.sync_copy(src_hbm.at[wid], src_v)
        pltpu.sync_copy(dst_hbm.at[wid], dst_v)

        pltpu.sync_copy(p_hbm.at[pl.ds(sid * rows_stage, rows_stage)],
                        p_s.at[pl.ds(sid * rows_stage, rows_stage)])

        plsc.subcore_barrier()

        gsems = sems[:nbuf]
        ssems = sems[nbuf:]

        def _start(j, b):
            pltpu.async_copy(p_s.at[src_v.at[j]], rows_v.at[b], gsems[b])

        for b in range(nbuf):
            _start(b, b)

        def _step(jo, carry):
            for b in range(nbuf):
                j = jo * nbuf + b
                pltpu.make_async_copy(
                    p_s.at[src_v.at[j]], rows_v.at[b], gsems[b]).wait()

                pltpu.async_copy(rows_v.at[b], acc.at[dst_v.at[j]], ssems[b],
                                 add=True)

                bp = (b - 1) % nbuf
                jp = j - 1

                @pl.when((jp >= 0) & (jp + nbuf < k_chunks))
                def _():
                    pltpu.make_async_copy(
                        rows_v.at[bp], acc.at[dst_v.at[jp]], ssems[bp]).wait()
                    _start(jp + nbuf, bp)
            return carry
        lax.fori_loop(0, k_chunks // nbuf, _step, 0)

        for b in range(nbuf):
            j = k_chunks - nbuf + b
            pltpu.make_async_copy(
                rows_v.at[b], acc.at[dst_v.at[j]], ssems[b]).wait()

        plsc.subcore_barrier()

        pltpu.sync_copy(acc.at[pl.ds(base, rows_per_tile)],
                        out_hbm.at[cid, pl.ds(base, rows_per_tile)])

    return agg


def kernel(x, edge_index, W1_0, b1_0, W2_0, b2_0, eps0, gamma0, beta0,
           W1_1, b1_1, W2_1, b2_1, eps1, gamma1, beta1):
    n, d_in = x.shape
    h0 = W1_0.shape[1]
    h1 = W1_1.shape[1]
    e = edge_index.shape[1]

    n_pad = ((n + 1 + 255) // 256) * 256
    k_chunks = -(-e // (NW * CHUNK))
    k_chunks = ((k_chunks + 1) // 2) * 2
    e_pad = NW * k_chunks * CHUNK

    src = edge_index[0]
    dst = edge_index[1]
    src3 = jnp.pad(src, (0, e_pad - e)).reshape(NW, k_chunks, CHUNK)
    dst3 = jnp.pad(dst, (0, e_pad - e),
                   constant_values=n).reshape(NW, k_chunks, CHUNK)

    inv = 1.0 / jnp.sqrt(1.0 + BN_EPS)
    g0 = gamma0 * inv
    w20s = W2_0 * g0[None, :]
    u0 = (g0 * b2_0 + beta0).reshape(1, h0)
    g1 = gamma1 * inv
    w21s = W2_1 * g1[None, :]
    v1 = (g1 * b2_1 + beta1).reshape(1, h1)
    bb0 = b1_0.reshape(1, h0)
    bb1 = b1_1.reshape(1, h1)
    c0 = (1.0 + eps0).reshape(1, 1)
    c1 = (1.0 + eps1).reshape(1, 1)

    blk = 2000
    grid = n // blk

    p0 = pl.pallas_call(
        _proj_body,
        grid=(grid,),
        in_specs=[pl.BlockSpec((blk, d_in), lambda i: (i, 0)),
                  pl.BlockSpec((d_in, h0), lambda i: (0, 0))],
        out_specs=pl.BlockSpec((blk, h0), lambda i: (i, 0)),
        out_shape=jax.ShapeDtypeStruct((n, h0), jnp.float32),
    )(x, W1_0)

    agg0 = _make_agg(n_pad, n, h0, k_chunks)(p0, src3, dst3)
    pa0 = agg0[0, :n]
    pb0 = agg0[1, :n]

    p1 = pl.pallas_call(
        _layer0_body,
        grid=(grid,),
        in_specs=[pl.BlockSpec((1, 1), lambda i: (0, 0)),
                  pl.BlockSpec((blk, h0), lambda i: (i, 0)),
                  pl.BlockSpec((blk, h0), lambda i: (i, 0)),
                  pl.BlockSpec((blk, h0), lambda i: (i, 0)),
                  pl.BlockSpec((1, h0), lambda i: (0, 0)),
                  pl.BlockSpec((h0, h0), lambda i: (0, 0)),
                  pl.BlockSpec((1, h0), lambda i: (0, 0)),
                  pl.BlockSpec((h0, h1), lambda i: (0, 0))],
        out_specs=pl.BlockSpec((blk, h1), lambda i: (i, 0)),
        out_shape=jax.ShapeDtypeStruct((n, h1), jnp.float32),
    )(c0, p0, pa0, pb0, bb0, w20s, u0, W1_1)

    agg1 = _make_agg(n_pad, n, h1, k_chunks)(p1, src3, dst3)
    pa1 = agg1[0, :n]
    pb1 = agg1[1, :n]

    n_inv = jnp.full((1, 1), 1.0 / n, jnp.float32)
    out = pl.pallas_call(
        _layer1_body,
        grid=(grid,),
        in_specs=[pl.BlockSpec((1, 1), lambda i: (0, 0)),
                  pl.BlockSpec((1, 1), lambda i: (0, 0)),
                  pl.BlockSpec((blk, h1), lambda i: (i, 0)),
                  pl.BlockSpec((blk, h1), lambda i: (i, 0)),
                  pl.BlockSpec((blk, h1), lambda i: (i, 0)),
                  pl.BlockSpec((1, h1), lambda i: (0, 0)),
                  pl.BlockSpec((h1, h1), lambda i: (0, 0)),
                  pl.BlockSpec((1, h1), lambda i: (0, 0))],
        out_specs=pl.BlockSpec((1, h1), lambda i: (0, 0)),
        out_shape=jax.ShapeDtypeStruct((1, h1), jnp.float32),
        scratch_shapes=[pltpu.VMEM((1, h1), jnp.float32)],
    )(n_inv, c1, p1, pa1, pb1, bb1, w21s, v1)

    return out


# speedup vs baseline: 2.3274x; 1.0649x over previous
import functools

import jax
import jax.numpy as jnp
from jax import lax
from jax.experimental import pallas as pl
from jax.experimental.pallas import tpu as pltpu
from jax.experimental.pallas import tpu_sc as plsc

BN_EPS = 1e-5
NC = 2
NS = 16
NW = NC * NS
CHUNK = 128


def _proj_body(x_ref, w_ref, o_ref):
    o_ref[...] = jnp.dot(x_ref[...], w_ref[...],
                         preferred_element_type=jnp.float32)


def _layer0_body(c0_ref, p_ref, pa_ref, pb_ref, bb0_ref, w20_ref, u0_ref,
                 w11_ref, o_ref):
    z = c0_ref[...] * p_ref[...] + pa_ref[...] + pb_ref[...] + bb0_ref[...]
    a = jnp.maximum(z, 0.0)
    h = jnp.maximum(
        jnp.dot(a, w20_ref[...], preferred_element_type=jnp.float32)
        + u0_ref[...], 0.0)
    o_ref[...] = jnp.dot(h, w11_ref[...], preferred_element_type=jnp.float32)


def _layer1_body(n_ref, c1_ref, p_ref, pa_ref, pb_ref, bb1_ref, w21_ref,
                 v1_ref, o_ref, acc_ref):
    i = pl.program_id(0)

    @pl.when(i == 0)
    def _():
        acc_ref[...] = jnp.zeros_like(acc_ref)

    z = c1_ref[...] * p_ref[...] + pa_ref[...] + pb_ref[...] + bb1_ref[...]
    q = jnp.maximum(z, 0.0)
    acc_ref[...] += jnp.sum(q, axis=0, keepdims=True)

    @pl.when(i == pl.num_programs(0) - 1)
    def _():
        mean = acc_ref[...] * n_ref[...]
        o_ref[...] = jnp.dot(mean, w21_ref[...],
                             preferred_element_type=jnp.float32) + v1_ref[...]


def _make_agg(n_pad, n_rows, feat, k_chunks):
    nbuf = 2
    rows_per_tile = n_pad // NS
    rows_stage = n_rows // NS
    zrows = max(d for d in range(1, 129)
                if rows_per_tile % d == 0 and d * feat <= 4800)
    assert k_chunks % nbuf == 0 and n_rows % NS == 0
    mesh = plsc.VectorSubcoreMesh(core_axis_name="c", subcore_axis_name="s")

    @functools.partial(
        pl.kernel,
        out_type=jax.ShapeDtypeStruct((NC * n_pad, feat), jnp.float32),
        mesh=mesh,
        scratch_types=[
            pltpu.VMEM((k_chunks, CHUNK), jnp.int32),
            pltpu.VMEM((k_chunks, CHUNK), jnp.int32),
            pltpu.VMEM((nbuf, CHUNK, feat), jnp.float32),
            pltpu.VMEM((zrows, feat), jnp.float32),
            pltpu.VMEM_SHARED((n_pad, feat), jnp.float32),
            pltpu.VMEM_SHARED((n_rows, feat), jnp.float32),
        ] + [pltpu.SemaphoreType.DMA] * (2 * nbuf),
        compiler_params=pltpu.CompilerParams(use_tc_tiling_on_sc=False),
    )
    def agg(p_hbm, src_hbm, dst_hbm, out_hbm, src_v, dst_v, rows_v, zb, acc,
            p_s, *sems):
        cid = lax.axis_index("c")
        sid = lax.axis_index("s")
        wid = cid * NS + sid
        base = sid * rows_per_tile

        def _zrow(r, carry):
            for c in range(feat // 16):
                zb[r, pl.ds(c * 16, 16)] = jnp.zeros((16,), jnp.float32)
            return carry
        lax.fori_loop(0, zrows, _zrow, 0)

        def _zcopy(t, carry):
            pltpu.sync_copy(zb, acc.at[pl.ds(base + t * zrows, zrows)])
            return carry
        lax.fori_loop(0, rows_per_tile // zrows, _zcopy, 0)

        pltpu.sync_copy(src_hbm.at[wid], src_v)
        pltpu.sync_copy(dst_hbm.at[wid], dst_v)

        pltpu.sync_copy(p_hbm.at[pl.ds(sid * rows_stage, rows_stage)],
                        p_s.at[pl.ds(sid * rows_stage, rows_stage)])

        plsc.subcore_barrier()

        gsems = sems[:nbuf]
        ssems = sems[nbuf:]

        def _start(j, b):
            pltpu.async_copy(p_s.at[src_v.at[j]], rows_v.at[b], gsems[b])

        for b in range(nbuf):
            _start(b, b)

        def _step(jo, carry):
            for b in range(nbuf):
                j = jo * nbuf + b
                pltpu.make_async_copy(
                    p_s.at[src_v.at[j]], rows_v.at[b], gsems[b]).wait()

                pltpu.async_copy(rows_v.at[b], acc.at[dst_v.at[j]], ssems[b],
                                 add=True)

                bp = (b - 1) % nbuf
                jp = j - 1

                @pl.when((jp >= 0) & (jp + nbuf < k_chunks))
                def _():
                    pltpu.make_async_copy(
                        rows_v.at[bp], acc.at[dst_v.at[jp]], ssems[bp]).wait()
                    _start(jp + nbuf, bp)
            return carry
        lax.fori_loop(0, k_chunks // nbuf, _step, 0)

        for b in range(nbuf):
            j = k_chunks - nbuf + b
            pltpu.make_async_copy(
                rows_v.at[b], acc.at[dst_v.at[j]], ssems[b]).wait()

        plsc.subcore_barrier()

        pltpu.sync_copy(acc.at[pl.ds(base, rows_per_tile)],
                        out_hbm.at[pl.ds(cid * n_pad + base, rows_per_tile)])

    return agg


def kernel(x, edge_index, W1_0, b1_0, W2_0, b2_0, eps0, gamma0, beta0,
           W1_1, b1_1, W2_1, b2_1, eps1, gamma1, beta1):
    n, d_in = x.shape
    h0 = W1_0.shape[1]
    h1 = W1_1.shape[1]
    e = edge_index.shape[1]

    blk = 2000
    n_pad = n + blk
    k_chunks = -(-e // (NW * CHUNK))
    k_chunks = ((k_chunks + 1) // 2) * 2
    e_pad = NW * k_chunks * CHUNK

    src = edge_index[0]
    dst = edge_index[1]
    src3 = jnp.pad(src, (0, e_pad - e)).reshape(NW, k_chunks, CHUNK)
    dst3 = jnp.pad(dst, (0, e_pad - e),
                   constant_values=n).reshape(NW, k_chunks, CHUNK)

    inv = 1.0 / jnp.sqrt(1.0 + BN_EPS)
    g0 = gamma0 * inv
    w20s = W2_0 * g0[None, :]
    u0 = (g0 * b2_0 + beta0).reshape(1, h0)
    g1 = gamma1 * inv
    w21s = W2_1 * g1[None, :]
    v1 = (g1 * b2_1 + beta1).reshape(1, h1)
    bb0 = b1_0.reshape(1, h0)
    bb1 = b1_1.reshape(1, h1)
    c0 = (1.0 + eps0).reshape(1, 1)
    c1 = (1.0 + eps1).reshape(1, 1)

    grid = n // blk
    npb = n_pad // blk

    p0 = pl.pallas_call(
        _proj_body,
        grid=(grid,),
        in_specs=[pl.BlockSpec((blk, d_in), lambda i: (i, 0)),
                  pl.BlockSpec((d_in, h0), lambda i: (0, 0))],
        out_specs=pl.BlockSpec((blk, h0), lambda i: (i, 0)),
        out_shape=jax.ShapeDtypeStruct((n, h0), jnp.float32),
    )(x, W1_0)

    agg0 = _make_agg(n_pad, n, h0, k_chunks)(p0, src3, dst3)

    p1 = pl.pallas_call(
        _layer0_body,
        grid=(grid,),
        in_specs=[pl.BlockSpec((1, 1), lambda i: (0, 0)),
                  pl.BlockSpec((blk, h0), lambda i: (i, 0)),
                  pl.BlockSpec((blk, h0), lambda i: (i, 0)),
                  pl.BlockSpec((blk, h0), lambda i: (npb + i, 0)),
                  pl.BlockSpec((1, h0), lambda i: (0, 0)),
                  pl.BlockSpec((h0, h0), lambda i: (0, 0)),
                  pl.BlockSpec((1, h0), lambda i: (0, 0)),
                  pl.BlockSpec((h0, h1), lambda i: (0, 0))],
        out_specs=pl.BlockSpec((blk, h1), lambda i: (i, 0)),
        out_shape=jax.ShapeDtypeStruct((n, h1), jnp.float32),
    )(c0, p0, agg0, agg0, bb0, w20s, u0, W1_1)

    agg1 = _make_agg(n_pad, n, h1, k_chunks)(p1, src3, dst3)

    n_inv = jnp.full((1, 1), 1.0 / n, jnp.float32)
    out = pl.pallas_call(
        _layer1_body,
        grid=(grid,),
        in_specs=[pl.BlockSpec((1, 1), lambda i: (0, 0)),
                  pl.BlockSpec((1, 1), lambda i: (0, 0)),
                  pl.BlockSpec((blk, h1), lambda i: (i, 0)),
                  pl.BlockSpec((blk, h1), lambda i: (i, 0)),
                  pl.BlockSpec((blk, h1), lambda i: (npb + i, 0)),
                  pl.BlockSpec((1, h1), lambda i: (0, 0)),
                  pl.BlockSpec((h1, h1), lambda i: (0, 0)),
                  pl.BlockSpec((1, h1), lambda i: (0, 0))],
        out_specs=pl.BlockSpec((1, h1), lambda i: (0, 0)),
        out_shape=jax.ShapeDtypeStruct((1, h1), jnp.float32),
        scratch_shapes=[pltpu.VMEM((1, h1), jnp.float32)],
    )(n_inv, c1, p1, agg1, agg1, bb1, w21s, v1)

    return out


# speedup vs baseline: 2.5911x; 1.1133x over previous
import functools

import jax
import jax.numpy as jnp
from jax import lax
from jax.experimental import pallas as pl
from jax.experimental.pallas import tpu as pltpu
from jax.experimental.pallas import tpu_sc as plsc

BN_EPS = 1e-5
NC = 2
NS = 16
NW = NC * NS
CHUNK = 128


def _proj_body(x_ref, w_ref, o_ref):
    o_ref[...] = jnp.dot(x_ref[...], w_ref[...],
                         preferred_element_type=jnp.float32)


def _layer0_body(c0_ref, p_ref, pa_ref, bb0_ref, w20_ref, u0_ref,
                 w11_ref, o_ref):
    h = bb0_ref.shape[-1]
    z = (c0_ref[...] * p_ref[...] + pa_ref[:, :h] + pa_ref[:, h:2 * h]
         + bb0_ref[...])
    a = jnp.maximum(z, 0.0)
    h = jnp.maximum(
        jnp.dot(a, w20_ref[...], preferred_element_type=jnp.float32)
        + u0_ref[...], 0.0)
    o_ref[...] = jnp.dot(h, w11_ref[...], preferred_element_type=jnp.float32)


def _layer1_body(n_ref, c1_ref, p_ref, pa_ref, bb1_ref, w21_ref,
                 v1_ref, o_ref, acc_ref):
    i = pl.program_id(0)

    @pl.when(i == 0)
    def _():
        acc_ref[...] = jnp.zeros_like(acc_ref)

    h = bb1_ref.shape[-1]
    z = (c1_ref[...] * p_ref[...] + pa_ref[:, :h] + pa_ref[:, h:2 * h]
         + bb1_ref[...])
    q = jnp.maximum(z, 0.0)
    acc_ref[...] += jnp.sum(q, axis=0, keepdims=True)

    @pl.when(i == pl.num_programs(0) - 1)
    def _():
        mean = acc_ref[...] * n_ref[...]
        o_ref[...] = jnp.dot(mean, w21_ref[...],
                             preferred_element_type=jnp.float32) + v1_ref[...]


def _make_agg(n_pad, n_rows, feat, k_chunks):
    nbuf = 2
    rows_per_tile = n_pad // NS
    rows_stage = n_rows // NS
    zrows = max(d for d in range(1, 129)
                if rows_per_tile % d == 0 and d * feat <= 4800)
    assert k_chunks % nbuf == 0 and n_rows % NS == 0
    mesh = plsc.VectorSubcoreMesh(core_axis_name="c", subcore_axis_name="s")

    @functools.partial(
        pl.kernel,
        out_type=jax.ShapeDtypeStruct((n_pad, 128), jnp.float32),
        mesh=mesh,
        scratch_types=[
            pltpu.VMEM((k_chunks, CHUNK), jnp.int32),
            pltpu.VMEM((k_chunks, CHUNK), jnp.int32),
            pltpu.VMEM((nbuf, CHUNK, feat), jnp.float32),
            pltpu.VMEM((zrows, feat), jnp.float32),
            pltpu.VMEM_SHARED((n_pad, feat), jnp.float32),
            pltpu.VMEM_SHARED((n_rows, feat), jnp.float32),
        ] + [pltpu.SemaphoreType.DMA] * (2 * nbuf),
        compiler_params=pltpu.CompilerParams(use_tc_tiling_on_sc=False),
    )
    def agg(p_hbm, src_hbm, dst_hbm, out_hbm, src_v, dst_v, rows_v, zb, acc,
            p_s, *sems):
        cid = lax.axis_index("c")
        sid = lax.axis_index("s")
        wid = cid * NS + sid
        base = sid * rows_per_tile

        def _zrow(r, carry):
            for c in range(feat // 16):
                zb[r, pl.ds(c * 16, 16)] = jnp.zeros((16,), jnp.float32)
            return carry
        lax.fori_loop(0, zrows, _zrow, 0)

        def _zcopy(t, carry):
            pltpu.sync_copy(zb, acc.at[pl.ds(base + t * zrows, zrows)])
            return carry
        lax.fori_loop(0, rows_per_tile // zrows, _zcopy, 0)

        pltpu.sync_copy(src_hbm.at[wid], src_v)
        pltpu.sync_copy(dst_hbm.at[wid], dst_v)

        pltpu.sync_copy(p_hbm.at[pl.ds(sid * rows_stage, rows_stage)],
                        p_s.at[pl.ds(sid * rows_stage, rows_stage)])

        plsc.subcore_barrier()

        gsems = sems[:nbuf]
        ssems = sems[nbuf:]

        def _start(j, b):
            pltpu.async_copy(p_s.at[src_v.at[j]], rows_v.at[b], gsems[b])

        for b in range(nbuf):
            _start(b, b)

        def _step(jo, carry):
            for b in range(nbuf):
                j = jo * nbuf + b
                pltpu.make_async_copy(
                    p_s.at[src_v.at[j]], rows_v.at[b], gsems[b]).wait()

                pltpu.async_copy(rows_v.at[b], acc.at[dst_v.at[j]], ssems[b],
                                 add=True)

                bp = (b - 1) % nbuf
                jp = j - 1

                @pl.when((jp >= 0) & (jp + nbuf < k_chunks))
                def _():
                    pltpu.make_async_copy(
                        rows_v.at[bp], acc.at[dst_v.at[jp]], ssems[bp]).wait()
                    _start(jp + nbuf, bp)
            return carry
        lax.fori_loop(0, k_chunks // nbuf, _step, 0)

        for b in range(nbuf):
            j = k_chunks - nbuf + b
            pltpu.make_async_copy(
                rows_v.at[b], acc.at[dst_v.at[j]], ssems[b]).wait()

        plsc.subcore_barrier()

        pltpu.sync_copy(
            acc.at[pl.ds(base, rows_per_tile)],
            out_hbm.at[pl.ds(base, rows_per_tile), pl.ds(cid * feat, feat)])

    return agg


def kernel(x, edge_index, W1_0, b1_0, W2_0, b2_0, eps0, gamma0, beta0,
           W1_1, b1_1, W2_1, b2_1, eps1, gamma1, beta1):
    n, d_in = x.shape
    h0 = W1_0.shape[1]
    h1 = W1_1.shape[1]
    e = edge_index.shape[1]

    blk = 2000
    n_pad = n + blk
    k_chunks = -(-e // (NW * CHUNK))
    k_chunks = ((k_chunks + 1) // 2) * 2
    e_pad = NW * k_chunks * CHUNK

    src = edge_index[0]
    dst = edge_index[1]
    src3 = jnp.pad(src, (0, e_pad - e)).reshape(NW, k_chunks, CHUNK)
    dst3 = jnp.pad(dst, (0, e_pad - e),
                   constant_values=n).reshape(NW, k_chunks, CHUNK)

    inv = 1.0 / jnp.sqrt(1.0 + BN_EPS)
    g0 = gamma0 * inv
    w20s = W2_0 * g0[None, :]
    u0 = (g0 * b2_0 + beta0).reshape(1, h0)
    g1 = gamma1 * inv
    w21s = W2_1 * g1[None, :]
    v1 = (g1 * b2_1 + beta1).reshape(1, h1)
    bb0 = b1_0.reshape(1, h0)
    bb1 = b1_1.reshape(1, h1)
    c0 = (1.0 + eps0).reshape(1, 1)
    c1 = (1.0 + eps1).reshape(1, 1)

    grid = n // blk
    npb = n_pad // blk

    p0 = pl.pallas_call(
        _proj_body,
        grid=(grid,),
        in_specs=[pl.BlockSpec((blk, d_in), lambda i: (i, 0)),
                  pl.BlockSpec((d_in, h0), lambda i: (0, 0))],
        out_specs=pl.BlockSpec((blk, h0), lambda i: (i, 0)),
        out_shape=jax.ShapeDtypeStruct((n, h0), jnp.float32),
    )(x, W1_0)

    agg0 = _make_agg(n_pad, n, h0, k_chunks)(p0, src3, dst3)

    p1 = pl.pallas_call(
        _layer0_body,
        grid=(grid,),
        in_specs=[pl.BlockSpec((1, 1), lambda i: (0, 0)),
                  pl.BlockSpec((blk, h0), lambda i: (i, 0)),
                  pl.BlockSpec((blk, 128), lambda i: (i, 0)),
                  pl.BlockSpec((1, h0), lambda i: (0, 0)),
                  pl.BlockSpec((h0, h0), lambda i: (0, 0)),
                  pl.BlockSpec((1, h0), lambda i: (0, 0)),
                  pl.BlockSpec((h0, h1), lambda i: (0, 0))],
        out_specs=pl.BlockSpec((blk, h1), lambda i: (i, 0)),
        out_shape=jax.ShapeDtypeStruct((n, h1), jnp.float32),
    )(c0, p0, agg0, bb0, w20s, u0, W1_1)

    agg1 = _make_agg(n_pad, n, h1, k_chunks)(p1, src3, dst3)

    n_inv = jnp.full((1, 1), 1.0 / n, jnp.float32)
    out = pl.pallas_call(
        _layer1_body,
        grid=(grid,),
        in_specs=[pl.BlockSpec((1, 1), lambda i: (0, 0)),
                  pl.BlockSpec((1, 1), lambda i: (0, 0)),
                  pl.BlockSpec((blk, h1), lambda i: (i, 0)),
                  pl.BlockSpec((blk, 128), lambda i: (i, 0)),
                  pl.BlockSpec((1, h1), lambda i: (0, 0)),
                  pl.BlockSpec((h1, h1), lambda i: (0, 0)),
                  pl.BlockSpec((1, h1), lambda i: (0, 0))],
        out_specs=pl.BlockSpec((1, h1), lambda i: (0, 0)),
        out_shape=jax.ShapeDtypeStruct((1, h1), jnp.float32),
        scratch_shapes=[pltpu.VMEM((1, h1), jnp.float32)],
    )(n_inv, c1, p1, agg1, bb1, w21s, v1)

    return out
